# baseline (speedup 1.0000x reference)
import functools

import jax
import jax.numpy as jnp
from jax.experimental import pallas as pl
from jax.experimental.pallas import tpu as pltpu
from jax.experimental.pallas import tpu_sc as plsc

N_EMBED = 1024
EMBED_DIM = 64
BETA = 0.25

N_TOKENS = 4 * 8 * 32 * 32
BLK_T = 2048
NUM_BLK = N_TOKENS // BLK_T


def _vq_body(zb_ref, emb_ref, embt2_ref,
             zq_ref, idx_ref, loss_ref, ppl_ref,
             esq_ref, counts_ref, lacc_ref):
    i = pl.program_id(0)
    emb = emb_ref[...]
    embt2 = embt2_ref[...]

    @pl.when(i == 0)
    def _init():
        esq_ref[...] = jnp.sum((0.5 * embt2) * (0.5 * embt2), axis=0)[None, :]
        counts_ref[...] = jnp.zeros_like(counts_ref)
        lacc_ref[...] = jnp.zeros_like(lacc_ref)
        loss_ref[...] = jnp.zeros_like(loss_ref)
        ppl_ref[...] = jnp.zeros_like(ppl_ref)

    zb = zb_ref[...]
    zsq = jnp.sum(zb * zb, axis=1, keepdims=True)
    mm2 = jnp.dot(zb, embt2, preferred_element_type=jnp.float32)
    d = (zsq + esq_ref[...]) - mm2

    dmin = jnp.min(d, axis=1, keepdims=True)
    iotaf = jax.lax.broadcasted_iota(
        jnp.int32, (1, N_EMBED), 1).astype(jnp.float32)
    idxf = jnp.min(jnp.where(d == dmin, iotaf, float(N_EMBED)), axis=1)
    enc = (iotaf == idxf[:, None]).astype(jnp.float32)
    idx = idxf.astype(jnp.int32)

    zq = jax.lax.dot_general(
        enc, emb, (((1,), (0,)), ((), ())),
        preferred_element_type=jnp.float32)
    zq_ref[...] = zb + (zq - zb)
    idx_ref[...] = idx[:, None]

    counts_ref[...] += jnp.sum(enc, axis=0)[None, :]
    lacc_ref[...] += jnp.sum((zq - zb) ** 2)[None, None]

    @pl.when(i == NUM_BLK - 1)
    def _fini():
        p = counts_ref[...] * (1.0 / N_TOKENS)
        ent = jnp.sum(p * jnp.log(p + 1e-10))
        ppl_ref[...] = jnp.exp(-ent)[None, None]
        loss_ref[...] = lacc_ref[...] * (BETA / (N_TOKENS * EMBED_DIM))


def _vq_call(z_flat, emb, embt2):
    return pl.pallas_call(
        _vq_body,
        grid=(NUM_BLK,),
        in_specs=[
            pl.BlockSpec((BLK_T, EMBED_DIM), lambda i: (i, 0)),
            pl.BlockSpec((N_EMBED, EMBED_DIM), lambda i: (0, 0)),
            pl.BlockSpec((EMBED_DIM, N_EMBED), lambda i: (0, 0)),
        ],
        out_specs=[
            pl.BlockSpec((BLK_T, EMBED_DIM), lambda i: (i, 0)),
            pl.BlockSpec((BLK_T, 1), lambda i: (i, 0)),
            pl.BlockSpec((1, 1), lambda i: (0, 0)),
            pl.BlockSpec((1, 1), lambda i: (0, 0)),
        ],
        out_shape=[
            jax.ShapeDtypeStruct((N_TOKENS, EMBED_DIM), jnp.float32),
            jax.ShapeDtypeStruct((N_TOKENS, 1), jnp.int32),
            jax.ShapeDtypeStruct((1, 1), jnp.float32),
            jax.ShapeDtypeStruct((1, 1), jnp.float32),
        ],
        scratch_shapes=[
            pltpu.VMEM((1, N_EMBED), jnp.float32),
            pltpu.VMEM((1, N_EMBED), jnp.float32),
            pltpu.VMEM((1, 1), jnp.float32),
        ],
    )(z_flat, emb, embt2)


_NC = 2
_NS = 16
_NW = _NC * _NS
_ROWS_PER_W = N_TOKENS // _NW
_ZROWS = 64
_ZCH = _ZROWS * N_EMBED
_NZ = _ROWS_PER_W // _ZROWS
_ENC_ELEMS = N_TOKENS * N_EMBED

_SC_MESH = plsc.VectorSubcoreMesh(core_axis_name="c", subcore_axis_name="s")


@functools.partial(
    pl.kernel, mesh=_SC_MESH, out_type=(),
    scratch_types=[
        pltpu.VMEM((_ZCH,), jnp.float32),
        pltpu.SemaphoreType.DMA,
    ])
def _sc_zero(enc_hbm, zsrc_hbm, zbuf, zsem):
    wid = jax.lax.axis_index("s") * _NC + jax.lax.axis_index("c")
    base = wid * _ROWS_PER_W * N_EMBED
    pltpu.sync_copy(zsrc_hbm, zbuf)
    copies = []
    for j in range(_NZ):
        copies.append(pltpu.async_copy(
            zbuf, enc_hbm.at[pl.ds(base + j * _ZCH, _ZCH)], zsem))
    for c in copies:
        c.wait()


@functools.partial(
    pl.kernel, mesh=_SC_MESH, out_type=(),
    scratch_types=[
        pltpu.VMEM((_ROWS_PER_W,), jnp.int32),
        pltpu.VMEM((8, 128), jnp.int32),
        pltpu.VMEM((128,), jnp.float32),
        pltpu.SemaphoreType.DMA,
    ])
def _sc_scatter(enc_hbm, idx_hbm, idxv, posv, onesv, ssem):
    wid = jax.lax.axis_index("s") * _NC + jax.lax.axis_index("c")
    tok0 = wid * _ROWS_PER_W
    pltpu.sync_copy(idx_hbm.at[pl.ds(tok0, _ROWS_PER_W)], idxv)
    lane = jax.lax.iota(jnp.int32, 16)
    for c in range(8):
        onesv[pl.ds(c * 16, 16)] = jnp.full((16,), 1.0, jnp.float32)
    for k in range(_ROWS_PER_W // 16):
        v = idxv[pl.ds(k * 16, 16)]
        pos = v + (tok0 + k * 16 + lane) * N_EMBED
        posv[k // 8, pl.ds((k % 8) * 16, 16)] = pos
    scats = []
    for r in range(8):
        scats.append(pltpu.async_copy(onesv, enc_hbm.at[posv.at[r]], ssem))
    for s in scats:
        s.wait()


def kernel(z, embedding_weight):
    b, c, dd, h, w = z.shape
    zp = jnp.transpose(z, (0, 2, 3, 4, 1))
    z_flat = zp.reshape(-1, c)
    embt2 = embedding_weight.T * 2.0

    enc_ref = jax.new_ref(jax.lax.empty((_ENC_ELEMS,), jnp.float32))
    zsrc = jnp.zeros((_ZCH,), jnp.float32)
    _sc_zero(enc_ref, zsrc)

    zq_st, idx2, loss2, ppl2 = _vq_call(z_flat, embedding_weight, embt2)

    _sc_scatter(enc_ref, idx2.reshape(N_TOKENS))
    enc = enc_ref[...].reshape(N_TOKENS, N_EMBED)

    z_q_out = jnp.transpose(zq_st.reshape(b, dd, h, w, c), (0, 4, 1, 2, 3))
    encoding_indices = idx2.reshape(N_TOKENS)
    return (z_q_out, loss2[0, 0], ppl2[0, 0], enc, encoding_indices)


# speedup vs baseline: 2.5350x; 2.5350x over previous
import jax
import jax.numpy as jnp
from jax.experimental import pallas as pl
from jax.experimental.pallas import tpu as pltpu

N_EMBED = 1024
EMBED_DIM = 64
BETA = 0.25

N_TOKENS = 4 * 8 * 32 * 32
N_SPATIAL = 8 * 32 * 32
BLK_T = 2048
BLK_PER_B = N_SPATIAL // BLK_T
NUM_BLK = N_TOKENS // BLK_T


def _vq_body(zb_ref, emb_ref, embt2_ref,
             enc_ref, zq_ref, idx_ref, loss_ref, ppl_ref,
             esq_ref, counts_ref, lacc_ref):
    i = pl.program_id(0)
    emb = emb_ref[...]
    embt2 = embt2_ref[...]

    @pl.when(i == 0)
    def _init():
        esq_ref[...] = jnp.sum((0.5 * embt2) * (0.5 * embt2), axis=0)[None, :]
        counts_ref[...] = jnp.zeros_like(counts_ref)
        lacc_ref[...] = jnp.zeros_like(lacc_ref)
        loss_ref[...] = jnp.zeros_like(loss_ref)
        ppl_ref[...] = jnp.zeros_like(ppl_ref)

    zb = zb_ref[...]
    zsq = jnp.sum(zb * zb, axis=1, keepdims=True)
    mm2 = jnp.dot(zb, embt2, preferred_element_type=jnp.float32)
    d = (zsq + esq_ref[...]) - mm2

    dmin = jnp.min(d, axis=1, keepdims=True)
    iotaf = jax.lax.broadcasted_iota(
        jnp.int32, (1, N_EMBED), 1).astype(jnp.float32)
    idxf = jnp.min(jnp.where(d == dmin, iotaf, float(N_EMBED)), axis=1)
    enc = (iotaf == idxf[:, None]).astype(jnp.float32)
    enc_ref[...] = enc
    idx = idxf.astype(jnp.int32)

    zq = jax.lax.dot_general(
        enc, emb, (((1,), (0,)), ((), ())),
        preferred_element_type=jnp.float32)
    zq_ref[0] = (zb + (zq - zb)).T
    idx_ref[...] = idx[:, None]

    counts_ref[...] += jnp.sum(enc, axis=0)[None, :]
    lacc_ref[...] += jnp.sum((zq - zb) ** 2)[None, None]

    @pl.when(i == NUM_BLK - 1)
    def _fini():
        p = counts_ref[...] * (1.0 / N_TOKENS)
        ent = jnp.sum(p * jnp.log(p + 1e-10))
        ppl_ref[...] = jnp.exp(-ent)[None, None]
        loss_ref[...] = lacc_ref[...] * (BETA / (N_TOKENS * EMBED_DIM))


def _vq_call(z_flat, emb, embt2):
    return pl.pallas_call(
        _vq_body,
        grid=(NUM_BLK,),
        in_specs=[
            pl.BlockSpec((BLK_T, EMBED_DIM), lambda i: (i, 0)),
            pl.BlockSpec((N_EMBED, EMBED_DIM), lambda i: (0, 0)),
            pl.BlockSpec((EMBED_DIM, N_EMBED), lambda i: (0, 0)),
        ],
        out_specs=[
            pl.BlockSpec((BLK_T, N_EMBED), lambda i: (i, 0)),
            pl.BlockSpec((1, EMBED_DIM, BLK_T),
                         lambda i: (i // BLK_PER_B, 0, i % BLK_PER_B)),
            pl.BlockSpec((BLK_T, 1), lambda i: (i, 0)),
            pl.BlockSpec((1, 1), lambda i: (0, 0)),
            pl.BlockSpec((1, 1), lambda i: (0, 0)),
        ],
        out_shape=[
            jax.ShapeDtypeStruct((N_TOKENS, N_EMBED), jnp.float32),
            jax.ShapeDtypeStruct((4, EMBED_DIM, N_SPATIAL), jnp.float32),
            jax.ShapeDtypeStruct((N_TOKENS, 1), jnp.int32),
            jax.ShapeDtypeStruct((1, 1), jnp.float32),
            jax.ShapeDtypeStruct((1, 1), jnp.float32),
        ],
        scratch_shapes=[
            pltpu.VMEM((1, N_EMBED), jnp.float32),
            pltpu.VMEM((1, N_EMBED), jnp.float32),
            pltpu.VMEM((1, 1), jnp.float32),
        ],
    )(z_flat, emb, embt2)


def kernel(z, embedding_weight):
    b, c, dd, h, w = z.shape
    zp = jnp.transpose(z, (0, 2, 3, 4, 1))
    z_flat = zp.reshape(-1, c)
    embt2 = embedding_weight.T * 2.0
    enc, zqc, idx2, loss2, ppl2 = _vq_call(z_flat, embedding_weight, embt2)
    z_q_out = zqc.reshape(b, c, dd, h, w)
    encoding_indices = idx2.reshape(N_TOKENS)
    return (z_q_out, loss2[0, 0], ppl2[0, 0], enc, encoding_indices)


# speedup vs baseline: 2.8792x; 1.1358x over previous
import jax
import jax.numpy as jnp
from jax.experimental import pallas as pl
from jax.experimental.pallas import tpu as pltpu

N_EMBED = 1024
EMBED_DIM = 64
BETA = 0.25

N_TOKENS = 4 * 8 * 32 * 32
N_SPATIAL = 8 * 32 * 32
BLK_T = 2048
BLK_PER_B = N_SPATIAL // BLK_T
NUM_BLK = N_TOKENS // BLK_T


def _vq_body(zb_ref, emb_ref, embt2_ref,
             enc_ref, zq_ref, idx_ref, loss_ref, ppl_ref,
             esq_ref, counts_ref, lacc_ref):
    i = pl.program_id(0)
    emb = emb_ref[...]
    embt2 = embt2_ref[...]

    @pl.when(i == 0)
    def _init():
        esq_ref[...] = jnp.sum((0.5 * embt2) * (0.5 * embt2), axis=0)[None, :]
        counts_ref[...] = jnp.zeros_like(counts_ref)
        lacc_ref[...] = jnp.zeros_like(lacc_ref)
        loss_ref[...] = jnp.zeros_like(loss_ref)
        ppl_ref[...] = jnp.zeros_like(ppl_ref)

    zb = zb_ref[...]
    zsq = jnp.sum(zb * zb, axis=1, keepdims=True)
    mm2 = jnp.dot(zb, embt2, preferred_element_type=jnp.float32)
    d = (zsq + esq_ref[...]) - mm2

    dmin = jnp.min(d, axis=1, keepdims=True)
    iotaf = jax.lax.broadcasted_iota(
        jnp.int32, (1, N_EMBED), 1).astype(jnp.float32)
    idxf = jnp.min(jnp.where(d == dmin, iotaf, float(N_EMBED)), axis=1)
    enc = (iotaf == idxf[:, None]).astype(jnp.float32)
    enc_ref[...] = enc
    idx = idxf.astype(jnp.int32)

    zq = jax.lax.dot_general(
        enc, emb, (((1,), (0,)), ((), ())),
        preferred_element_type=jnp.float32)
    zq_ref[...] = zb + (zq - zb)
    idx_ref[...] = idx[:, None]

    counts_ref[...] += jnp.sum(enc, axis=0)[None, :]
    lacc_ref[...] += jnp.sum((zq - zb) ** 2)[None, None]

    @pl.when(i == NUM_BLK - 1)
    def _fini():
        p = counts_ref[...] * (1.0 / N_TOKENS)
        ent = jnp.sum(p * jnp.log(p + 1e-10))
        ppl_ref[...] = jnp.exp(-ent)[None, None]
        loss_ref[...] = lacc_ref[...] * (BETA / (N_TOKENS * EMBED_DIM))


def _vq_call(z_flat, emb, embt2):
    return pl.pallas_call(
        _vq_body,
        grid=(NUM_BLK,),
        in_specs=[
            pl.BlockSpec((BLK_T, EMBED_DIM), lambda i: (i, 0)),
            pl.BlockSpec((N_EMBED, EMBED_DIM), lambda i: (0, 0)),
            pl.BlockSpec((EMBED_DIM, N_EMBED), lambda i: (0, 0)),
        ],
        out_specs=[
            pl.BlockSpec((BLK_T, N_EMBED), lambda i: (i, 0)),
            pl.BlockSpec((BLK_T, EMBED_DIM), lambda i: (i, 0)),
            pl.BlockSpec((BLK_T, 1), lambda i: (i, 0)),
            pl.BlockSpec((1, 1), lambda i: (0, 0)),
            pl.BlockSpec((1, 1), lambda i: (0, 0)),
        ],
        out_shape=[
            jax.ShapeDtypeStruct((N_TOKENS, N_EMBED), jnp.float32),
            jax.ShapeDtypeStruct((N_TOKENS, EMBED_DIM), jnp.float32),
            jax.ShapeDtypeStruct((N_TOKENS, 1), jnp.int32),
            jax.ShapeDtypeStruct((1, 1), jnp.float32),
            jax.ShapeDtypeStruct((1, 1), jnp.float32),
        ],
        scratch_shapes=[
            pltpu.VMEM((1, N_EMBED), jnp.float32),
            pltpu.VMEM((1, N_EMBED), jnp.float32),
            pltpu.VMEM((1, 1), jnp.float32),
        ],
    )(z_flat, emb, embt2)


def kernel(z, embedding_weight):
    b, c, dd, h, w = z.shape
    zp = jnp.transpose(z, (0, 2, 3, 4, 1))
    z_flat = zp.reshape(-1, c)
    embt2 = embedding_weight.T * 2.0
    enc, zq_st, idx2, loss2, ppl2 = _vq_call(z_flat, embedding_weight, embt2)
    z_q_out = jnp.transpose(zq_st.reshape(b, dd, h, w, c), (0, 4, 1, 2, 3))
    encoding_indices = idx2.reshape(N_TOKENS)
    return (z_q_out, loss2[0, 0], ppl2[0, 0], enc, encoding_indices)
